# Initial kernel scaffold; baseline (speedup 1.0000x reference)
import jax
import jax.numpy as jnp
from jax.experimental import pallas as pl


def kernel(x, W_gate):
    raise NotImplementedError("write your pallas kernel here")


# speedup vs baseline: 4.9917x; 4.9917x over previous
import functools

import jax
import jax.numpy as jnp
from jax.experimental import pallas as pl
from jax.experimental.pallas import tpu as pltpu

HIDDEN = 4096
NUM_EXPERTS = 64
TOP_K = 8
TOKEN_TILE = 512


def _router_kernel(x_ref, w_ref, probs_ref, map_ref):
    x = x_ref[...]
    w = w_ref[...]
    logits = jax.lax.dot_general(
        x, w, (((1,), (1,)), ((), ())),
        preferred_element_type=jnp.float32,
    )
    n = logits.shape[0]
    col = jax.lax.broadcasted_iota(jnp.int32, (n, NUM_EXPERTS), 1)
    neg_inf = jnp.float32(-jnp.inf)

    selected = jnp.zeros((n, NUM_EXPERTS), dtype=jnp.bool_)
    for _ in range(TOP_K):
        avail = jnp.where(selected, neg_inf, logits)
        m = jnp.max(avail, axis=1, keepdims=True)
        is_max = avail == m
        first = jnp.min(jnp.where(is_max, col, NUM_EXPERTS), axis=1, keepdims=True)
        selected = jnp.logical_or(selected, col == first)

    rowmax = jnp.max(logits, axis=1, keepdims=True)
    e = jnp.exp(logits - rowmax)
    denom = jnp.sum(jnp.where(selected, e, 0.0), axis=1, keepdims=True)
    probs_ref[...] = e / denom
    map_ref[...] = selected.astype(jnp.float32)


@functools.partial(jax.jit, static_argnames=())
def kernel(x, W_gate):
    n_tokens = x.shape[0]
    grid = (n_tokens // TOKEN_TILE,)
    probs, map_f32 = pl.pallas_call(
        _router_kernel,
        grid=grid,
        in_specs=[
            pl.BlockSpec((TOKEN_TILE, HIDDEN), lambda i: (i, 0)),
            pl.BlockSpec((NUM_EXPERTS, HIDDEN), lambda i: (0, 0)),
        ],
        out_specs=[
            pl.BlockSpec((TOKEN_TILE, NUM_EXPERTS), lambda i: (i, 0)),
            pl.BlockSpec((TOKEN_TILE, NUM_EXPERTS), lambda i: (i, 0)),
        ],
        out_shape=[
            jax.ShapeDtypeStruct((n_tokens, NUM_EXPERTS), jnp.float32),
            jax.ShapeDtypeStruct((n_tokens, NUM_EXPERTS), jnp.float32),
        ],
        compiler_params=pltpu.CompilerParams(
            dimension_semantics=("parallel",),
        ),
    )(x, W_gate)
    return probs, map_f32.astype(jnp.bool_)


# speedup vs baseline: 6.5334x; 1.3089x over previous
import functools

import jax
import jax.numpy as jnp
from jax.experimental import pallas as pl
from jax.experimental.pallas import tpu as pltpu

HIDDEN = 4096
NUM_EXPERTS = 64
TOP_K = 8
TOKEN_TILE = 1024


def _router_kernel(x_ref, w_ref, probs_ref, map_ref):
    x = x_ref[...]
    w = w_ref[...]
    logits = jax.lax.dot_general(
        x, w, (((1,), (1,)), ((), ())),
        preferred_element_type=jnp.float32,
    )
    n = logits.shape[0]
    neg_inf = jnp.float32(-jnp.inf)

    selected = jnp.zeros((n, NUM_EXPERTS), dtype=jnp.bool_)
    rowmax = None
    denom = None
    for _ in range(TOP_K):
        avail = jnp.where(selected, neg_inf, logits)
        m = jnp.max(avail, axis=1, keepdims=True)
        selected = jnp.logical_or(selected, avail == m)
        if rowmax is None:
            rowmax = m
            denom = jnp.ones_like(m)
        else:
            denom = denom + jnp.exp(m - rowmax)

    e = jnp.exp(logits - rowmax)
    probs_ref[...] = e * (1.0 / denom)
    map_ref[...] = selected.astype(jnp.float32)


@functools.partial(jax.jit, static_argnames=())
def kernel(x, W_gate):
    n_tokens = x.shape[0]
    grid = (n_tokens // TOKEN_TILE,)
    probs, map_f32 = pl.pallas_call(
        _router_kernel,
        grid=grid,
        in_specs=[
            pl.BlockSpec((TOKEN_TILE, HIDDEN), lambda i: (i, 0)),
            pl.BlockSpec((NUM_EXPERTS, HIDDEN), lambda i: (0, 0)),
        ],
        out_specs=[
            pl.BlockSpec((TOKEN_TILE, NUM_EXPERTS), lambda i: (i, 0)),
            pl.BlockSpec((TOKEN_TILE, NUM_EXPERTS), lambda i: (i, 0)),
        ],
        out_shape=[
            jax.ShapeDtypeStruct((n_tokens, NUM_EXPERTS), jnp.float32),
            jax.ShapeDtypeStruct((n_tokens, NUM_EXPERTS), jnp.float32),
        ],
        compiler_params=pltpu.CompilerParams(
            dimension_semantics=("parallel",),
        ),
    )(x, W_gate)
    return probs, map_f32.astype(jnp.bool_)
